# baseline (speedup 1.0000x reference)
import functools

import jax
import jax.numpy as jnp
from jax import lax
from jax.experimental import pallas as pl
from jax.experimental.pallas import tpu as pltpu
from jax.experimental.pallas import tpu_sc as plsc

B = 16384
N_ENT = 1000000
ENT_DIM = 64
REL_DIM = 64
N_REL = 1000
OUT_DIM = ENT_DIM + REL_DIM + ENT_DIM

NC = 2
NS = 16
NW = NC * NS
ROWS_PER_W = B // NW
CHUNK = 128
N_CHUNKS = ROWS_PER_W // CHUNK
L = 16

TR_BR = 2048
PACK_H = 245 * TR_BR
TR_STEPS = PACK_H // TR_BR


def _tc_transpose_body(xlo_ref, xhi_ref, o_ref):
    o_ref[:, 0:ENT_DIM] = xlo_ref[...].T
    o_ref[:, ENT_DIM:2 * ENT_DIM] = xhi_ref[...].T


def _tc_transpose(entT):
    return pl.pallas_call(
        _tc_transpose_body,
        grid=(TR_STEPS,),
        in_specs=[
            pl.BlockSpec((ENT_DIM, TR_BR), lambda i: (0, i)),
            pl.BlockSpec((ENT_DIM, TR_BR),
                         lambda i: (0, jnp.minimum(i + TR_STEPS,
                                                   N_ENT // TR_BR))),
        ],
        out_specs=pl.BlockSpec((TR_BR, 2 * ENT_DIM), lambda i: (i, 0)),
        out_shape=jax.ShapeDtypeStruct((PACK_H, 2 * ENT_DIM), jnp.float32),
    )(entT, entT)


def _rsqrt_newton(x):
    i = lax.bitcast_convert_type(x, jnp.int32)
    i = jnp.int32(0x5F3759DF) - lax.shift_right_arithmetic(i, jnp.int32(1))
    y = lax.bitcast_convert_type(i, jnp.float32)
    half_x = x * jnp.float32(0.5)
    for _ in range(2):
        y = y * (jnp.float32(1.5) - half_x * y * y)
    return y


_GATHER_DNUMS = lax.GatherDimensionNumbers(
    offset_dims=(), collapsed_slice_dims=(0,), start_index_map=(0,))


def _lane_shuffle(v, idx):
    return lax.gather(v, idx[:, None], _GATHER_DNUMS, (1,),
                      mode=lax.GatherScatterMode.PROMISE_IN_BOUNDS)


def _body(h_hbm, r_hbm, t_hbm, ent_hbm, rel_hbm, out_hbm,
          idx_h, idx_r, idx_t, rows_h, rows_t, rel_v, out_v, sem):
    wid = lax.axis_index("s") * NC + lax.axis_index("c")
    w_base = wid * ROWS_PER_W

    pltpu.sync_copy(rel_hbm, rel_v)

    def chunk_body(ci, _):
        base = w_base + ci * CHUNK
        pltpu.sync_copy(h_hbm.at[pl.ds(base, CHUNK)], idx_h)
        pltpu.sync_copy(t_hbm.at[pl.ds(base, CHUNK)], idx_t)
        pltpu.sync_copy(r_hbm.at[pl.ds(base, CHUNK)], idx_r)

        def fire(g, _):
            gbase = g * L
            iv_h = idx_h[pl.ds(gbase, L)]
            iv_t = idx_t[pl.ds(gbase, L)]
            lo_h = lax.shift_right_logical(iv_h - jnp.int32(PACK_H),
                                           jnp.int32(31))
            lo_t = lax.shift_right_logical(iv_t - jnp.int32(PACK_H),
                                           jnp.int32(31))
            ivs_h = iv_h - jnp.int32(PACK_H) + lo_h * jnp.int32(PACK_H)
            ivs_t = iv_t - jnp.int32(PACK_H) + lo_t * jnp.int32(PACK_H)
            for k in range(L):
                j = gbase + k
                pltpu.async_copy(ent_hbm.at[pl.ds(ivs_h[k], 1)],
                                 rows_h.at[pl.ds(j, 1)], sem)
                pltpu.async_copy(ent_hbm.at[pl.ds(ivs_t[k], 1)],
                                 rows_t.at[pl.ds(j, 1)], sem)
            return 0

        lax.fori_loop(0, CHUNK // L, fire, 0)
        pltpu.make_async_copy(ent_hbm.at[pl.ds(0, CHUNK)], rows_h, sem).wait()
        pltpu.make_async_copy(ent_hbm.at[pl.ds(0, CHUNK)], rows_t, sem).wait()

        lanes = lax.iota(jnp.int32, L)

        @plsc.parallel_loop(0, CHUNK // L, step=1)
        def group_body(g):
            gbase = g * L
            ivs_r = idx_r[pl.ds(gbase, L)]
            hi_h = (jnp.int32(1) - lax.shift_right_logical(
                idx_h[pl.ds(gbase, L)] - jnp.int32(PACK_H), jnp.int32(31))
            ) * jnp.int32(ENT_DIM)
            hi_t = (jnp.int32(1) - lax.shift_right_logical(
                idx_t[pl.ds(gbase, L)] - jnp.int32(PACK_H), jnp.int32(31))
            ) * jnp.int32(ENT_DIM)
            for k in range(L):
                j = gbase + k
                rel_base = ivs_r[k] * jnp.int32(REL_DIM)
                off_h = hi_h[k]
                off_t = hi_t[k]
                xs = []
                acc = jnp.zeros((L,), jnp.float32)
                for src, off in ((rows_h, off_h), (rows_t, off_t)):
                    for c in range(ENT_DIM // L):
                        x = src[j, pl.ds(off + c * L, L)]
                        xs.append(x)
                        acc = acc + x * x
                for c in range(REL_DIM // L):
                    x = rel_v[pl.ds(rel_base + c * L, L)]
                    xs.append(x)
                    acc = acc + x * x
                for s in (8, 4, 2, 1):
                    acc = acc + _lane_shuffle(acc, lanes ^ s)
                invv = _rsqrt_newton(jnp.maximum(acc, jnp.float32(1e-24)))
                for c in range(ENT_DIM // L):
                    out_v[j, pl.ds(c * L, L)] = xs[c] * invv
                for c in range(REL_DIM // L):
                    out_v[j, pl.ds(ENT_DIM + c * L, L)] = xs[8 + c] * invv
                for c in range(ENT_DIM // L):
                    out_v[j, pl.ds(ENT_DIM + REL_DIM + c * L, L)] = \
                        xs[4 + c] * invv

        pltpu.sync_copy(out_v, out_hbm.at[pl.ds(base, CHUNK)])
        return 0

    lax.fori_loop(0, N_CHUNKS, chunk_body, 0)


def kernel(h, r, t, ent_weight, rel_weight):
    ent_rm = _tc_transpose(ent_weight.T)
    k = functools.partial(
        pl.kernel,
        out_type=jax.ShapeDtypeStruct((B, OUT_DIM), jnp.float32),
        mesh=plsc.VectorSubcoreMesh(core_axis_name="c", subcore_axis_name="s"),
        compiler_params=pltpu.CompilerParams(use_tc_tiling_on_sc=True),
        scratch_types=[
            pltpu.VMEM((CHUNK,), jnp.int32),
            pltpu.VMEM((CHUNK,), jnp.int32),
            pltpu.VMEM((CHUNK,), jnp.int32),
            pltpu.VMEM((CHUNK, 2 * ENT_DIM), jnp.float32),
            pltpu.VMEM((CHUNK, 2 * ENT_DIM), jnp.float32),
            pltpu.VMEM((REL_DIM * N_REL,), jnp.float32),
            pltpu.VMEM((CHUNK, OUT_DIM), jnp.float32),
            pltpu.SemaphoreType.DMA,
        ],
    )(_body)
    return k(h.astype(jnp.int32), r.astype(jnp.int32), t.astype(jnp.int32),
             ent_rm, rel_weight.reshape(-1))


# speedup vs baseline: 1.3256x; 1.3256x over previous
import functools

import jax
import jax.numpy as jnp
from jax import lax
from jax.experimental import pallas as pl
from jax.experimental.pallas import tpu as pltpu
from jax.experimental.pallas import tpu_sc as plsc

B = 16384
N_ENT = 1000000
ENT_DIM = 64
REL_DIM = 64
N_REL = 1000
OUT_DIM = ENT_DIM + REL_DIM + ENT_DIM

NC = 2
NS = 16
NW = NC * NS
ROWS_PER_W = B // NW
CHUNK = 128
N_CHUNKS = ROWS_PER_W // CHUNK
L = 16

TR_BR = 8192
PACK_H = 62 * TR_BR
TR_STEPS = PACK_H // TR_BR


def _tc_transpose_body(xlo_ref, xhi_ref, o_ref):
    o_ref[:, 0:ENT_DIM] = xlo_ref[...].T
    o_ref[:, ENT_DIM:2 * ENT_DIM] = xhi_ref[...].T


def _tc_transpose(entT):
    return pl.pallas_call(
        _tc_transpose_body,
        grid=(TR_STEPS,),
        in_specs=[
            pl.BlockSpec((ENT_DIM, TR_BR), lambda i: (0, i)),
            pl.BlockSpec((ENT_DIM, TR_BR),
                         lambda i: (0, jnp.minimum(i + TR_STEPS,
                                                   N_ENT // TR_BR))),
        ],
        out_specs=pl.BlockSpec((TR_BR, 2 * ENT_DIM), lambda i: (i, 0)),
        out_shape=jax.ShapeDtypeStruct((PACK_H, 2 * ENT_DIM), jnp.float32),
    )(entT, entT)


def _rsqrt_newton(x):
    i = lax.bitcast_convert_type(x, jnp.int32)
    i = jnp.int32(0x5F3759DF) - lax.shift_right_arithmetic(i, jnp.int32(1))
    y = lax.bitcast_convert_type(i, jnp.float32)
    half_x = x * jnp.float32(0.5)
    for _ in range(2):
        y = y * (jnp.float32(1.5) - half_x * y * y)
    return y


_GATHER_DNUMS = lax.GatherDimensionNumbers(
    offset_dims=(), collapsed_slice_dims=(0,), start_index_map=(0,))


def _lane_shuffle(v, idx):
    return lax.gather(v, idx[:, None], _GATHER_DNUMS, (1,),
                      mode=lax.GatherScatterMode.PROMISE_IN_BOUNDS)


def _body(h_hbm, r_hbm, t_hbm, ent_hbm, rel_hbm, out_hbm,
          idx_h, idx_r, idx_t, rows_h, rows_t, rel_v, out_v, sem):
    wid = lax.axis_index("s") * NC + lax.axis_index("c")
    w_base = wid * ROWS_PER_W

    pltpu.sync_copy(rel_hbm, rel_v)

    def chunk_body(ci, _):
        base = w_base + ci * CHUNK
        pltpu.sync_copy(h_hbm.at[pl.ds(base, CHUNK)], idx_h)
        pltpu.sync_copy(t_hbm.at[pl.ds(base, CHUNK)], idx_t)
        pltpu.sync_copy(r_hbm.at[pl.ds(base, CHUNK)], idx_r)

        def fire(g, _):
            gbase = g * L
            iv_h = idx_h[pl.ds(gbase, L)]
            iv_t = idx_t[pl.ds(gbase, L)]
            lo_h = lax.shift_right_logical(iv_h - jnp.int32(PACK_H),
                                           jnp.int32(31))
            lo_t = lax.shift_right_logical(iv_t - jnp.int32(PACK_H),
                                           jnp.int32(31))
            ivs_h = iv_h - jnp.int32(PACK_H) + lo_h * jnp.int32(PACK_H)
            ivs_t = iv_t - jnp.int32(PACK_H) + lo_t * jnp.int32(PACK_H)
            for k in range(L):
                j = gbase + k
                pltpu.async_copy(ent_hbm.at[pl.ds(ivs_h[k], 1)],
                                 rows_h.at[pl.ds(j, 1)], sem)
                pltpu.async_copy(ent_hbm.at[pl.ds(ivs_t[k], 1)],
                                 rows_t.at[pl.ds(j, 1)], sem)
            return 0

        lax.fori_loop(0, CHUNK // L, fire, 0)
        pltpu.make_async_copy(ent_hbm.at[pl.ds(0, CHUNK)], rows_h, sem).wait()
        pltpu.make_async_copy(ent_hbm.at[pl.ds(0, CHUNK)], rows_t, sem).wait()

        lanes = lax.iota(jnp.int32, L)

        @plsc.parallel_loop(0, CHUNK // L, step=1)
        def group_body(g):
            gbase = g * L
            ivs_r = idx_r[pl.ds(gbase, L)]
            hi_h = (jnp.int32(1) - lax.shift_right_logical(
                idx_h[pl.ds(gbase, L)] - jnp.int32(PACK_H), jnp.int32(31))
            ) * jnp.int32(ENT_DIM)
            hi_t = (jnp.int32(1) - lax.shift_right_logical(
                idx_t[pl.ds(gbase, L)] - jnp.int32(PACK_H), jnp.int32(31))
            ) * jnp.int32(ENT_DIM)
            for k in range(L):
                j = gbase + k
                rel_base = ivs_r[k] * jnp.int32(REL_DIM)
                off_h = hi_h[k]
                off_t = hi_t[k]
                xs = []
                acc = jnp.zeros((L,), jnp.float32)
                for src, off in ((rows_h, off_h), (rows_t, off_t)):
                    for c in range(ENT_DIM // L):
                        x = src[j, pl.ds(off + c * L, L)]
                        xs.append(x)
                        acc = acc + x * x
                for c in range(REL_DIM // L):
                    x = rel_v[pl.ds(rel_base + c * L, L)]
                    xs.append(x)
                    acc = acc + x * x
                for s in (8, 4, 2, 1):
                    acc = acc + _lane_shuffle(acc, lanes ^ s)
                invv = _rsqrt_newton(jnp.maximum(acc, jnp.float32(1e-24)))
                for c in range(ENT_DIM // L):
                    out_v[j, pl.ds(c * L, L)] = xs[c] * invv
                for c in range(REL_DIM // L):
                    out_v[j, pl.ds(ENT_DIM + c * L, L)] = xs[8 + c] * invv
                for c in range(ENT_DIM // L):
                    out_v[j, pl.ds(ENT_DIM + REL_DIM + c * L, L)] = \
                        xs[4 + c] * invv

        pltpu.sync_copy(out_v, out_hbm.at[pl.ds(base, CHUNK)])
        return 0

    lax.fori_loop(0, N_CHUNKS, chunk_body, 0)


def kernel(h, r, t, ent_weight, rel_weight):
    ent_rm = _tc_transpose(ent_weight.T)
    k = functools.partial(
        pl.kernel,
        out_type=jax.ShapeDtypeStruct((B, OUT_DIM), jnp.float32),
        mesh=plsc.VectorSubcoreMesh(core_axis_name="c", subcore_axis_name="s"),
        compiler_params=pltpu.CompilerParams(use_tc_tiling_on_sc=True),
        scratch_types=[
            pltpu.VMEM((CHUNK,), jnp.int32),
            pltpu.VMEM((CHUNK,), jnp.int32),
            pltpu.VMEM((CHUNK,), jnp.int32),
            pltpu.VMEM((CHUNK, 2 * ENT_DIM), jnp.float32),
            pltpu.VMEM((CHUNK, 2 * ENT_DIM), jnp.float32),
            pltpu.VMEM((REL_DIM * N_REL,), jnp.float32),
            pltpu.VMEM((CHUNK, OUT_DIM), jnp.float32),
            pltpu.SemaphoreType.DMA,
        ],
    )(_body)
    return k(h.astype(jnp.int32), r.astype(jnp.int32), t.astype(jnp.int32),
             ent_rm, rel_weight.reshape(-1))


# speedup vs baseline: 1.3878x; 1.0469x over previous
import functools

import jax
import jax.numpy as jnp
from jax import lax
from jax.experimental import pallas as pl
from jax.experimental.pallas import tpu as pltpu
from jax.experimental.pallas import tpu_sc as plsc

B = 16384
N_ENT = 1000000
ENT_DIM = 64
REL_DIM = 64
N_REL = 1000
OUT_DIM = ENT_DIM + REL_DIM + ENT_DIM

NC = 2
NS = 16
NW = NC * NS
ROWS_PER_W = B // NW
CHUNK = 128
N_CHUNKS = ROWS_PER_W // CHUNK
L = 16

TR_BR = 16384
PACK_H = 31 * TR_BR
TR_STEPS = PACK_H // TR_BR


def _tc_transpose_body(xlo_ref, xhi_ref, o_ref):
    o_ref[:, 0:ENT_DIM] = xlo_ref[...].T
    o_ref[:, ENT_DIM:2 * ENT_DIM] = xhi_ref[...].T


def _tc_transpose(entT):
    return pl.pallas_call(
        _tc_transpose_body,
        grid=(TR_STEPS,),
        in_specs=[
            pl.BlockSpec((ENT_DIM, TR_BR), lambda i: (0, i)),
            pl.BlockSpec((ENT_DIM, TR_BR),
                         lambda i: (0, jnp.minimum(i + TR_STEPS,
                                                   N_ENT // TR_BR))),
        ],
        out_specs=pl.BlockSpec((TR_BR, 2 * ENT_DIM), lambda i: (i, 0)),
        out_shape=jax.ShapeDtypeStruct((PACK_H, 2 * ENT_DIM), jnp.float32),
    )(entT, entT)


def _rsqrt_newton(x):
    i = lax.bitcast_convert_type(x, jnp.int32)
    i = jnp.int32(0x5F3759DF) - lax.shift_right_arithmetic(i, jnp.int32(1))
    y = lax.bitcast_convert_type(i, jnp.float32)
    half_x = x * jnp.float32(0.5)
    for _ in range(2):
        y = y * (jnp.float32(1.5) - half_x * y * y)
    return y


_GATHER_DNUMS = lax.GatherDimensionNumbers(
    offset_dims=(), collapsed_slice_dims=(0,), start_index_map=(0,))


def _lane_shuffle(v, idx):
    return lax.gather(v, idx[:, None], _GATHER_DNUMS, (1,),
                      mode=lax.GatherScatterMode.PROMISE_IN_BOUNDS)


def _body(h_hbm, r_hbm, t_hbm, ent_hbm, rel_hbm, out_hbm,
          idx_h, idx_r, idx_t, rows_h, rows_t, rel_v, out_v, sem):
    wid = lax.axis_index("s") * NC + lax.axis_index("c")
    w_base = wid * ROWS_PER_W

    pltpu.sync_copy(rel_hbm, rel_v)

    def chunk_body(ci, _):
        base = w_base + ci * CHUNK
        pltpu.sync_copy(h_hbm.at[pl.ds(base, CHUNK)], idx_h)
        pltpu.sync_copy(t_hbm.at[pl.ds(base, CHUNK)], idx_t)
        pltpu.sync_copy(r_hbm.at[pl.ds(base, CHUNK)], idx_r)

        def fire(g, _):
            gbase = g * L
            iv_h = idx_h[pl.ds(gbase, L)]
            iv_t = idx_t[pl.ds(gbase, L)]
            lo_h = lax.shift_right_logical(iv_h - jnp.int32(PACK_H),
                                           jnp.int32(31))
            lo_t = lax.shift_right_logical(iv_t - jnp.int32(PACK_H),
                                           jnp.int32(31))
            ivs_h = iv_h - jnp.int32(PACK_H) + lo_h * jnp.int32(PACK_H)
            ivs_t = iv_t - jnp.int32(PACK_H) + lo_t * jnp.int32(PACK_H)
            for k in range(L):
                j = gbase + k
                pltpu.async_copy(ent_hbm.at[pl.ds(ivs_h[k], 1)],
                                 rows_h.at[pl.ds(j, 1)], sem)
                pltpu.async_copy(ent_hbm.at[pl.ds(ivs_t[k], 1)],
                                 rows_t.at[pl.ds(j, 1)], sem)
            return 0

        lax.fori_loop(0, CHUNK // L, fire, 0)
        pltpu.make_async_copy(ent_hbm.at[pl.ds(0, CHUNK)], rows_h, sem).wait()
        pltpu.make_async_copy(ent_hbm.at[pl.ds(0, CHUNK)], rows_t, sem).wait()

        lanes = lax.iota(jnp.int32, L)

        @plsc.parallel_loop(0, CHUNK // L, step=1)
        def group_body(g):
            gbase = g * L
            ivs_r = idx_r[pl.ds(gbase, L)]
            hi_h = (jnp.int32(1) - lax.shift_right_logical(
                idx_h[pl.ds(gbase, L)] - jnp.int32(PACK_H), jnp.int32(31))
            ) * jnp.int32(ENT_DIM)
            hi_t = (jnp.int32(1) - lax.shift_right_logical(
                idx_t[pl.ds(gbase, L)] - jnp.int32(PACK_H), jnp.int32(31))
            ) * jnp.int32(ENT_DIM)
            for k in range(L):
                j = gbase + k
                rel_base = ivs_r[k] * jnp.int32(REL_DIM)
                off_h = hi_h[k]
                off_t = hi_t[k]
                xs = []
                acc = jnp.zeros((L,), jnp.float32)
                for src, off in ((rows_h, off_h), (rows_t, off_t)):
                    for c in range(ENT_DIM // L):
                        x = src[j, pl.ds(off + c * L, L)]
                        xs.append(x)
                        acc = acc + x * x
                for c in range(REL_DIM // L):
                    x = rel_v[pl.ds(rel_base + c * L, L)]
                    xs.append(x)
                    acc = acc + x * x
                for s in (8, 4, 2, 1):
                    acc = acc + _lane_shuffle(acc, lanes ^ s)
                invv = _rsqrt_newton(jnp.maximum(acc, jnp.float32(1e-24)))
                for c in range(ENT_DIM // L):
                    out_v[j, pl.ds(c * L, L)] = xs[c] * invv
                for c in range(REL_DIM // L):
                    out_v[j, pl.ds(ENT_DIM + c * L, L)] = xs[8 + c] * invv
                for c in range(ENT_DIM // L):
                    out_v[j, pl.ds(ENT_DIM + REL_DIM + c * L, L)] = \
                        xs[4 + c] * invv

        pltpu.sync_copy(out_v, out_hbm.at[pl.ds(base, CHUNK)])
        return 0

    lax.fori_loop(0, N_CHUNKS, chunk_body, 0)


def kernel(h, r, t, ent_weight, rel_weight):
    ent_rm = _tc_transpose(ent_weight.T)
    k = functools.partial(
        pl.kernel,
        out_type=jax.ShapeDtypeStruct((B, OUT_DIM), jnp.float32),
        mesh=plsc.VectorSubcoreMesh(core_axis_name="c", subcore_axis_name="s"),
        compiler_params=pltpu.CompilerParams(use_tc_tiling_on_sc=True),
        scratch_types=[
            pltpu.VMEM((CHUNK,), jnp.int32),
            pltpu.VMEM((CHUNK,), jnp.int32),
            pltpu.VMEM((CHUNK,), jnp.int32),
            pltpu.VMEM((CHUNK, 2 * ENT_DIM), jnp.float32),
            pltpu.VMEM((CHUNK, 2 * ENT_DIM), jnp.float32),
            pltpu.VMEM((REL_DIM * N_REL,), jnp.float32),
            pltpu.VMEM((CHUNK, OUT_DIM), jnp.float32),
            pltpu.SemaphoreType.DMA,
        ],
    )(_body)
    return k(h.astype(jnp.int32), r.astype(jnp.int32), t.astype(jnp.int32),
             ent_rm, rel_weight.reshape(-1))
